# Initial kernel scaffold; baseline (speedup 1.0000x reference)
import jax
import jax.numpy as jnp
from jax.experimental import pallas as pl


def kernel(node_features, node_positions, atom_type_ids, segment_ids):
    raise NotImplementedError("write your pallas kernel here")


# speedup vs baseline: 5.8089x; 5.8089x over previous
import functools

import jax
import jax.numpy as jnp
from jax import lax
from jax.experimental import pallas as pl
from jax.experimental.pallas import tpu as pltpu
from jax.experimental.pallas import tpu_sc as plsc

N_AT = 320000
R = 10000
H = 128
NW = 32
CHUNK = 256
NCHUNKS = N_AT // CHUNK
ITERS = -(-NCHUNKS // NW)
ACC_ROWS = 10112
ROWS_PER_TILE = ACC_ROWS // 16
TRASH = R
NSTAT = 11
LWORDS = ACC_ROWS * NSTAT
LROWS = 896
LPAD = LROWS * 128


def _prep_body(seg_ref, type_ref, px_ref, py_ref, pz_ref, adj_ref, *st_refs):
    seg = seg_ref[...]
    t = type_ref[...]
    px = px_ref[...]
    py = py_ref[...]
    pz = pz_ref[...]
    one = jnp.ones_like(px)
    zero = jnp.zeros_like(px)
    isN = jnp.where(t == 0, one, zero)
    isCA = jnp.where(t == 1, one, zero)
    isC = jnp.where(t == 2, one, zero)
    isCB = jnp.where(t == 4, one, zero)
    del zero
    adj_ref[...] = jnp.where(t == 1, seg, jnp.full_like(seg, TRASH))
    vals = (one, isN, isCA, isC, isCB,
            isCA * px, isCA * py, isCA * pz,
            isCB * px, isCB * py, isCB * pz)
    for ref, v in zip(st_refs, vals):
        ref[...] = v


def _make_sc_scatter(width):
    def body(rows_hbm, key3_hbm, zero_hbm, out_hbm, tile, idx, acc):
        c = lax.axis_index("c")
        s = lax.axis_index("s")
        w = c * 16 + s
        rows0 = s * ROWS_PER_TILE

        pltpu.sync_copy(zero_hbm.at[pl.ds(rows0, ROWS_PER_TILE)],
                        acc.at[pl.ds(rows0, ROWS_PER_TILE)])
        plsc.subcore_barrier()

        def it_body(i, carry):
            chunk = w + i * NW

            @pl.when(chunk < NCHUNKS)
            def _():
                pltpu.sync_copy(rows_hbm.at[pl.ds(chunk * CHUNK, CHUNK)],
                                tile)
                pltpu.sync_copy(key3_hbm.at[chunk], idx)
                for g in range(CHUNK // 128):
                    pltpu.sync_copy(tile.at[pl.ds(g * 128, 128)],
                                    acc.at[idx.at[g]], add=True)
            return carry

        lax.fori_loop(0, ITERS, it_body, 0)
        plsc.subcore_barrier()

        out0 = c * ACC_ROWS + rows0
        pltpu.sync_copy(acc.at[pl.ds(rows0, ROWS_PER_TILE)],
                        out_hbm.at[pl.ds(out0, ROWS_PER_TILE)])

    mesh = plsc.VectorSubcoreMesh(core_axis_name="c", subcore_axis_name="s")
    return functools.partial(
        pl.kernel,
        out_type=[jax.ShapeDtypeStruct((2 * ACC_ROWS, width), jnp.float32)],
        mesh=mesh,
        scratch_types=[
            pltpu.VMEM((CHUNK, width), jnp.float32),
            pltpu.VMEM((8, 128), jnp.int32),
            pltpu.VMEM_SHARED((ACC_ROWS, width), jnp.float32),
        ],
    )(body)


def _sc_stats_scatter(rows16_hbm, key3_hbm, ztile_hbm, zero_hbm, out_hbm,
                      tile, idx, acc):
    c = lax.axis_index("c")
    s = lax.axis_index("s")
    w = c * 16 + s
    rows0 = s * ROWS_PER_TILE

    pltpu.sync_copy(ztile_hbm, tile)
    pltpu.sync_copy(zero_hbm.at[pl.ds(rows0, ROWS_PER_TILE)],
                    acc.at[pl.ds(rows0, ROWS_PER_TILE)])
    plsc.subcore_barrier()

    def it_body(i, carry):
        chunk = w + i * NW

        @pl.when(chunk < NCHUNKS)
        def _():
            pltpu.sync_copy(rows16_hbm.at[pl.ds(chunk * CHUNK, CHUNK)],
                            tile.at[:, pl.ds(0, 16)])
            pltpu.sync_copy(key3_hbm.at[chunk], idx)
            for g in range(CHUNK // 128):
                pltpu.sync_copy(tile.at[pl.ds(g * 128, 128)],
                                acc.at[idx.at[g]], add=True)
        return carry

    lax.fori_loop(0, ITERS, it_body, 0)
    plsc.subcore_barrier()

    out0 = c * ACC_ROWS + rows0
    pltpu.sync_copy(acc.at[pl.ds(rows0, ROWS_PER_TILE)],
                    out_hbm.at[pl.ds(out0, ROWS_PER_TILE)])


def _finish_body(f0_ref, f1_ref, s0_ref, s1_ref, feat_ref, misc_ref):
    st = s0_ref[...] + s1_ref[...]
    cnt = st[:, 0:1]
    cN = st[:, 1:2]
    cCA = st[:, 2:3]
    cC = st[:, 3:4]
    cCB = st[:, 4:5]
    dCA = jnp.maximum(cCA, 1.0)
    feat_ref[...] = (f0_ref[...] + f1_ref[...]) / dCA

    pax = st[:, 5:6] / dCA
    pay = st[:, 6:7] / dCA
    paz = st[:, 7:8] / dCA
    dCB = jnp.maximum(cCB, 1.0)
    pbx = st[:, 8:9] / dCB
    pby = st[:, 9:10] / dCB
    pbz = st[:, 10:11] / dCB
    no_cb = (jnp.abs(pbx) + jnp.abs(pby) + jnp.abs(pbz)) < 1e-6
    pbx = jnp.where(no_cb, pax, pbx)
    pby = jnp.where(no_cb, pay, pby)
    pbz = jnp.where(no_cb, paz, pbz)

    e1x = pbx - pax
    e1y = pby - pay
    e1z = pbz - paz
    n1 = jnp.sqrt(e1x * e1x + e1y * e1y + e1z * e1z)
    d1 = jnp.maximum(n1, 1e-6)
    e1ux = e1x / d1
    e1uy = e1y / d1
    e1uz = e1z / d1
    n2a = jnp.sqrt(e1ux * e1ux + e1uy * e1uy)
    use_b = n2a < 1e-6
    e2x = jnp.where(use_b, -e1uz, e1uy)
    e2y = jnp.where(use_b, 0.0, -e1ux)
    e2z = jnp.where(use_b, e1ux, 0.0)
    n2 = jnp.sqrt(e2x * e2x + e2y * e2y + e2z * e2z)
    d2 = jnp.maximum(n2, 1e-6)
    e2ux = e2x / d2
    e2uy = e2y / d2
    e2uz = e2z / d2
    e3x = e1uy * e2uz - e1uz * e2uy
    e3y = e1uz * e2ux - e1ux * e2uz
    e3z = e1ux * e2uy - e1uy * e2ux

    blk = f0_ref.shape[0]
    ridx = (lax.broadcasted_iota(jnp.int32, (blk, 1), 0)
            + pl.program_id(0) * blk)
    valid = (n1 > 1e-6) & (n2 > 1e-6) & (ridx < R - 1)

    maskf = jnp.where((cnt >= 3.0) & (cN > 0.0) & (cCA > 0.0) & (cC > 0.0),
                      1.0, 0.0)
    misc_ref[:, 0:1] = pax
    misc_ref[:, 1:2] = pay
    misc_ref[:, 2:3] = paz
    misc_ref[:, 3:4] = pbx
    misc_ref[:, 4:5] = pby
    misc_ref[:, 5:6] = pbz
    misc_ref[:, 6:7] = jnp.where(valid, e1ux, 1.0)
    misc_ref[:, 7:8] = jnp.where(valid, e2ux, 0.0)
    misc_ref[:, 8:9] = jnp.where(valid, e3x, 0.0)
    misc_ref[:, 9:10] = jnp.where(valid, e1uy, 0.0)
    misc_ref[:, 10:11] = jnp.where(valid, e2uy, 1.0)
    misc_ref[:, 11:12] = jnp.where(valid, e3y, 0.0)
    misc_ref[:, 12:13] = jnp.where(valid, e1uz, 0.0)
    misc_ref[:, 13:14] = jnp.where(valid, e2uz, 0.0)
    misc_ref[:, 14:15] = jnp.where(valid, e3z, 1.0)
    misc_ref[:, 15:16] = maskf


def kernel(node_features, node_positions, atom_type_ids, segment_ids):
    SUB = 2560
    PAD = SUB * 128 - N_AT
    segs1 = segment_ids.astype(jnp.int32)
    types1 = atom_type_ids.astype(jnp.int32)
    segs2 = jnp.pad(segs1, (0, PAD)).reshape(SUB, 128)
    types2 = jnp.pad(types1, (0, PAD), constant_values=21).reshape(SUB, 128)
    pos_t = jnp.pad(node_positions.astype(jnp.float32).T,
                    ((0, 0), (0, PAD))).reshape(3, SUB, 128)

    B2 = 320
    grid0 = SUB // B2
    spec2 = pl.BlockSpec((B2, 128), lambda i: (i, 0))
    prep_out = pl.pallas_call(
        _prep_body,
        grid=(grid0,),
        in_specs=[spec2] * 5,
        out_specs=[spec2] * (1 + NSTAT),
        out_shape=([jax.ShapeDtypeStruct((SUB, 128), jnp.int32)]
                   + [jax.ShapeDtypeStruct((SUB, 128), jnp.float32)] * NSTAT),
    )(segs2, types2, pos_t[0], pos_t[1], pos_t[2])
    adj = prep_out[0]

    stats_wide = jnp.pad(
        jnp.stack(prep_out[1:], axis=0).reshape(NSTAT, SUB * 128)[:, :N_AT].T,
        ((0, 0), (0, H - NSTAT)))
    KR = CHUNK // 128
    adj3 = jnp.pad(adj.reshape(SUB * 128)[:N_AT].reshape(NCHUNKS, KR, 128),
                   ((0, 0), (0, 8 - KR), (0, 0)))
    seg3 = jnp.pad(segs1.reshape(NCHUNKS, KR, 128),
                   ((0, 0), (0, 8 - KR), (0, 0)))
    zf = jnp.zeros((ACC_ROWS, H), jnp.float32)
    ztile = jnp.zeros((CHUNK, H), jnp.float32)

    feat_part, = _make_sc_scatter(H)(node_features, adj3, zf)
    stat_part, = _make_sc_scatter(H)(stats_wide, seg3, zf)

    f0 = feat_part[:R]
    f1 = feat_part[ACC_ROWS:ACC_ROWS + R]
    s0 = stat_part[:R]
    s1 = stat_part[ACC_ROWS:ACC_ROWS + R]

    BLK = 1000
    grid = R // BLK
    res_feat, misc = pl.pallas_call(
        _finish_body,
        grid=(grid,),
        in_specs=[pl.BlockSpec((BLK, H), lambda i: (i, 0))] * 4,
        out_specs=[
            pl.BlockSpec((BLK, H), lambda i: (i, 0)),
            pl.BlockSpec((BLK, 16), lambda i: (i, 0)),
        ],
        out_shape=[
            jax.ShapeDtypeStruct((R, H), jnp.float32),
            jax.ShapeDtypeStruct((R, 16), jnp.float32),
        ],
    )(f0, f1, s0, s1)

    pos_CA = misc[:, 0:3]
    pos_CB = misc[:, 3:6]
    frames = misc[:, 6:15].reshape(R, 3, 3)
    residue_mask = misc[:, 15] > 0.5
    return (res_feat, pos_CA, pos_CB, frames, segment_ids, residue_mask)


# speedup vs baseline: 6.5303x; 1.1242x over previous
import functools

import jax
import jax.numpy as jnp
from jax import lax
from jax.experimental import pallas as pl
from jax.experimental.pallas import tpu as pltpu
from jax.experimental.pallas import tpu_sc as plsc

N_AT = 320000
R = 10000
H = 128
NW = 32
CHUNK = 128
NCHUNKS = N_AT // CHUNK
ITERS = 80
ACC_ROWS = 10112
ROWS_PER_TILE = ACC_ROWS // 16
TRASH = R
NSTAT = 11
LWORDS = ACC_ROWS * NSTAT
LROWS = 896
LPAD = LROWS * 128


def _prep_body(seg_ref, type_ref, px_ref, py_ref, pz_ref, adj_ref, *st_refs):
    seg = seg_ref[...]
    t = type_ref[...]
    px = px_ref[...]
    py = py_ref[...]
    pz = pz_ref[...]
    one = jnp.ones_like(px)
    zero = jnp.zeros_like(px)
    isN = jnp.where(t == 0, one, zero)
    isCA = jnp.where(t == 1, one, zero)
    isC = jnp.where(t == 2, one, zero)
    isCB = jnp.where(t == 4, one, zero)
    del zero
    adj_ref[...] = jnp.where(t == 1, seg, jnp.full_like(seg, TRASH))
    vals = (one, isN, isCA, isC, isCB,
            isCA * px, isCA * py, isCA * pz,
            isCB * px, isCB * py, isCB * pz)
    for ref, v in zip(st_refs, vals):
        ref[...] = v


def _make_sc_scatter(width):
    def body(rows_hbm, key3_hbm, zero_hbm, out_hbm,
             tile0, tile1, idx0, idx1, acc, st0, si0, st1, si1):
        c = lax.axis_index("c")
        s = lax.axis_index("s")
        w = c * 16 + s
        rows0 = s * ROWS_PER_TILE

        pltpu.sync_copy(zero_hbm.at[pl.ds(rows0, ROWS_PER_TILE)],
                        acc.at[pl.ds(rows0, ROWS_PER_TILE)])
        plsc.subcore_barrier()

        bufs = ((tile0, idx0, st0, si0), (tile1, idx1, st1, si1))

        def start(chunk, tile, idx, st, si):
            pltpu.async_copy(rows_hbm.at[pl.ds(chunk * CHUNK, CHUNK)],
                             tile, st)
            pltpu.async_copy(key3_hbm.at[chunk], idx, si)

        def wait(chunk, tile, idx, st, si):
            pltpu.make_async_copy(rows_hbm.at[pl.ds(chunk * CHUNK, CHUNK)],
                                  tile, st).wait()
            pltpu.make_async_copy(key3_hbm.at[chunk], idx, si).wait()

        def step(chunk, mine, other):
            @pl.when(chunk < NCHUNKS)
            def _():
                wait(chunk, *mine)

                @pl.when(chunk + NW < NCHUNKS)
                def _():
                    start(chunk + NW, *other)

                pltpu.sync_copy(mine[0], acc.at[mine[1].at[0]], add=True)

        @pl.when(w < NCHUNKS)
        def _():
            start(w, *bufs[0])

        def pair_body(j, carry):
            c0 = w + (2 * j) * NW
            step(c0, bufs[0], bufs[1])
            step(c0 + NW, bufs[1], bufs[0])
            return carry

        lax.fori_loop(0, ITERS // 2, pair_body, 0)
        plsc.subcore_barrier()

        out0 = c * ACC_ROWS + rows0
        pltpu.sync_copy(acc.at[pl.ds(rows0, ROWS_PER_TILE)],
                        out_hbm.at[pl.ds(out0, ROWS_PER_TILE)])

    mesh = plsc.VectorSubcoreMesh(core_axis_name="c", subcore_axis_name="s")
    return functools.partial(
        pl.kernel,
        out_type=[jax.ShapeDtypeStruct((2 * ACC_ROWS, width), jnp.float32)],
        mesh=mesh,
        scratch_types=[
            pltpu.VMEM((CHUNK, width), jnp.float32),
            pltpu.VMEM((CHUNK, width), jnp.float32),
            pltpu.VMEM((8, 128), jnp.int32),
            pltpu.VMEM((8, 128), jnp.int32),
            pltpu.VMEM_SHARED((ACC_ROWS, width), jnp.float32),
            pltpu.SemaphoreType.DMA,
            pltpu.SemaphoreType.DMA,
            pltpu.SemaphoreType.DMA,
            pltpu.SemaphoreType.DMA,
        ],
    )(body)


def _finish_body(f0_ref, f1_ref, s0_ref, s1_ref, feat_ref, misc_ref):
    st = s0_ref[...] + s1_ref[...]
    cnt = st[:, 0:1]
    cN = st[:, 1:2]
    cCA = st[:, 2:3]
    cC = st[:, 3:4]
    cCB = st[:, 4:5]
    dCA = jnp.maximum(cCA, 1.0)
    feat_ref[...] = (f0_ref[...] + f1_ref[...]) / dCA

    pax = st[:, 5:6] / dCA
    pay = st[:, 6:7] / dCA
    paz = st[:, 7:8] / dCA
    dCB = jnp.maximum(cCB, 1.0)
    pbx = st[:, 8:9] / dCB
    pby = st[:, 9:10] / dCB
    pbz = st[:, 10:11] / dCB
    no_cb = (jnp.abs(pbx) + jnp.abs(pby) + jnp.abs(pbz)) < 1e-6
    pbx = jnp.where(no_cb, pax, pbx)
    pby = jnp.where(no_cb, pay, pby)
    pbz = jnp.where(no_cb, paz, pbz)

    e1x = pbx - pax
    e1y = pby - pay
    e1z = pbz - paz
    n1 = jnp.sqrt(e1x * e1x + e1y * e1y + e1z * e1z)
    d1 = jnp.maximum(n1, 1e-6)
    e1ux = e1x / d1
    e1uy = e1y / d1
    e1uz = e1z / d1
    n2a = jnp.sqrt(e1ux * e1ux + e1uy * e1uy)
    use_b = n2a < 1e-6
    e2x = jnp.where(use_b, -e1uz, e1uy)
    e2y = jnp.where(use_b, 0.0, -e1ux)
    e2z = jnp.where(use_b, e1ux, 0.0)
    n2 = jnp.sqrt(e2x * e2x + e2y * e2y + e2z * e2z)
    d2 = jnp.maximum(n2, 1e-6)
    e2ux = e2x / d2
    e2uy = e2y / d2
    e2uz = e2z / d2
    e3x = e1uy * e2uz - e1uz * e2uy
    e3y = e1uz * e2ux - e1ux * e2uz
    e3z = e1ux * e2uy - e1uy * e2ux

    blk = f0_ref.shape[0]
    ridx = (lax.broadcasted_iota(jnp.int32, (blk, 1), 0)
            + pl.program_id(0) * blk)
    valid = (n1 > 1e-6) & (n2 > 1e-6) & (ridx < R - 1)

    maskf = jnp.where((cnt >= 3.0) & (cN > 0.0) & (cCA > 0.0) & (cC > 0.0),
                      1.0, 0.0)
    misc_ref[:, 0:1] = pax
    misc_ref[:, 1:2] = pay
    misc_ref[:, 2:3] = paz
    misc_ref[:, 3:4] = pbx
    misc_ref[:, 4:5] = pby
    misc_ref[:, 5:6] = pbz
    misc_ref[:, 6:7] = jnp.where(valid, e1ux, 1.0)
    misc_ref[:, 7:8] = jnp.where(valid, e2ux, 0.0)
    misc_ref[:, 8:9] = jnp.where(valid, e3x, 0.0)
    misc_ref[:, 9:10] = jnp.where(valid, e1uy, 0.0)
    misc_ref[:, 10:11] = jnp.where(valid, e2uy, 1.0)
    misc_ref[:, 11:12] = jnp.where(valid, e3y, 0.0)
    misc_ref[:, 12:13] = jnp.where(valid, e1uz, 0.0)
    misc_ref[:, 13:14] = jnp.where(valid, e2uz, 0.0)
    misc_ref[:, 14:15] = jnp.where(valid, e3z, 1.0)
    misc_ref[:, 15:16] = maskf


def kernel(node_features, node_positions, atom_type_ids, segment_ids):
    SUB = 2560
    PAD = SUB * 128 - N_AT
    segs1 = segment_ids.astype(jnp.int32)
    types1 = atom_type_ids.astype(jnp.int32)
    segs2 = jnp.pad(segs1, (0, PAD)).reshape(SUB, 128)
    types2 = jnp.pad(types1, (0, PAD), constant_values=21).reshape(SUB, 128)
    pos_t = jnp.pad(node_positions.astype(jnp.float32).T,
                    ((0, 0), (0, PAD))).reshape(3, SUB, 128)

    B2 = 320
    grid0 = SUB // B2
    spec2 = pl.BlockSpec((B2, 128), lambda i: (i, 0))
    prep_out = pl.pallas_call(
        _prep_body,
        grid=(grid0,),
        in_specs=[spec2] * 5,
        out_specs=[spec2] * (1 + NSTAT),
        out_shape=([jax.ShapeDtypeStruct((SUB, 128), jnp.int32)]
                   + [jax.ShapeDtypeStruct((SUB, 128), jnp.float32)] * NSTAT),
    )(segs2, types2, pos_t[0], pos_t[1], pos_t[2])
    adj = prep_out[0]

    stats_wide = jnp.pad(
        jnp.stack(prep_out[1:], axis=0).reshape(NSTAT, SUB * 128)[:, :N_AT].T,
        ((0, 0), (0, H - NSTAT)))
    KR = CHUNK // 128
    adj3 = jnp.pad(adj.reshape(SUB * 128)[:N_AT].reshape(NCHUNKS, KR, 128),
                   ((0, 0), (0, 8 - KR), (0, 0)))
    seg3 = jnp.pad(segs1.reshape(NCHUNKS, KR, 128),
                   ((0, 0), (0, 8 - KR), (0, 0)))
    zf = jnp.zeros((ACC_ROWS, H), jnp.float32)

    feat_part, = _make_sc_scatter(H)(node_features, adj3, zf)
    stat_part, = _make_sc_scatter(H)(stats_wide, seg3, zf)

    f0 = feat_part[:R]
    f1 = feat_part[ACC_ROWS:ACC_ROWS + R]
    s0 = stat_part[:R]
    s1 = stat_part[ACC_ROWS:ACC_ROWS + R]

    BLK = 1000
    grid = R // BLK
    res_feat, misc = pl.pallas_call(
        _finish_body,
        grid=(grid,),
        in_specs=[pl.BlockSpec((BLK, H), lambda i: (i, 0))] * 4,
        out_specs=[
            pl.BlockSpec((BLK, H), lambda i: (i, 0)),
            pl.BlockSpec((BLK, 16), lambda i: (i, 0)),
        ],
        out_shape=[
            jax.ShapeDtypeStruct((R, H), jnp.float32),
            jax.ShapeDtypeStruct((R, 16), jnp.float32),
        ],
    )(f0, f1, s0, s1)

    pos_CA = misc[:, 0:3]
    pos_CB = misc[:, 3:6]
    frames = misc[:, 6:15].reshape(R, 3, 3)
    residue_mask = misc[:, 15] > 0.5
    return (res_feat, pos_CA, pos_CB, frames, segment_ids, residue_mask)


# speedup vs baseline: 6.9068x; 1.0577x over previous
import functools

import jax
import jax.numpy as jnp
from jax import lax
from jax.experimental import pallas as pl
from jax.experimental.pallas import tpu as pltpu
from jax.experimental.pallas import tpu_sc as plsc

N_AT = 320000
R = 10000
H = 128
NW = 32
CHUNK = 128
NCHUNKS = N_AT // CHUNK
ITERS = 80
ACC_ROWS = 10112
ROWS_PER_TILE = ACC_ROWS // 16
TRASH = R
NSTAT = 11
LWORDS = ACC_ROWS * NSTAT
LROWS = 896
LPAD = LROWS * 128


def _prep_body(seg_ref, type_ref, px_ref, py_ref, pz_ref, adj_ref, *st_refs):
    seg = seg_ref[...]
    t = type_ref[...]
    px = px_ref[...]
    py = py_ref[...]
    pz = pz_ref[...]
    one = jnp.ones_like(px)
    zero = jnp.zeros_like(px)
    isN = jnp.where(t == 0, one, zero)
    isCA = jnp.where(t == 1, one, zero)
    isC = jnp.where(t == 2, one, zero)
    isCB = jnp.where(t == 4, one, zero)
    del zero
    lane = lax.broadcasted_iota(jnp.int32, t.shape, 1)
    adj_ref[...] = jnp.where(t == 1, seg, TRASH + lane % (ACC_ROWS - TRASH))
    vals = (one, isN, isCA, isC, isCB,
            isCA * px, isCA * py, isCA * pz,
            isCB * px, isCB * py, isCB * pz)
    for ref, v in zip(st_refs, vals):
        ref[...] = v


def _make_sc_scatter(width):
    def body(rows_hbm, key3_hbm, zero_hbm, out_hbm,
             tile0, tile1, idx0, idx1, acc, st0, si0, st1, si1):
        c = lax.axis_index("c")
        s = lax.axis_index("s")
        w = c * 16 + s
        rows0 = s * ROWS_PER_TILE

        pltpu.sync_copy(zero_hbm.at[pl.ds(rows0, ROWS_PER_TILE)],
                        acc.at[pl.ds(rows0, ROWS_PER_TILE)])
        plsc.subcore_barrier()

        bufs = ((tile0, idx0, st0, si0), (tile1, idx1, st1, si1))

        def start(chunk, tile, idx, st, si):
            pltpu.async_copy(rows_hbm.at[pl.ds(chunk * CHUNK, CHUNK)],
                             tile, st)
            pltpu.async_copy(key3_hbm.at[chunk], idx, si)

        def wait(chunk, tile, idx, st, si):
            pltpu.make_async_copy(rows_hbm.at[pl.ds(chunk * CHUNK, CHUNK)],
                                  tile, st).wait()
            pltpu.make_async_copy(key3_hbm.at[chunk], idx, si).wait()

        def step(chunk, mine, other):
            @pl.when(chunk < NCHUNKS)
            def _():
                wait(chunk, *mine)

                @pl.when(chunk + NW < NCHUNKS)
                def _():
                    start(chunk + NW, *other)

                pltpu.sync_copy(mine[0], acc.at[mine[1].at[0]], add=True)

        @pl.when(w < NCHUNKS)
        def _():
            start(w, *bufs[0])

        def pair_body(j, carry):
            c0 = w + (2 * j) * NW
            step(c0, bufs[0], bufs[1])
            step(c0 + NW, bufs[1], bufs[0])
            return carry

        lax.fori_loop(0, ITERS // 2, pair_body, 0)
        plsc.subcore_barrier()

        out0 = c * ACC_ROWS + rows0
        pltpu.sync_copy(acc.at[pl.ds(rows0, ROWS_PER_TILE)],
                        out_hbm.at[pl.ds(out0, ROWS_PER_TILE)])

    mesh = plsc.VectorSubcoreMesh(core_axis_name="c", subcore_axis_name="s")
    return functools.partial(
        pl.kernel,
        out_type=[jax.ShapeDtypeStruct((2 * ACC_ROWS, width), jnp.float32)],
        mesh=mesh,
        scratch_types=[
            pltpu.VMEM((CHUNK, width), jnp.float32),
            pltpu.VMEM((CHUNK, width), jnp.float32),
            pltpu.VMEM((8, 128), jnp.int32),
            pltpu.VMEM((8, 128), jnp.int32),
            pltpu.VMEM_SHARED((ACC_ROWS, width), jnp.float32),
            pltpu.SemaphoreType.DMA,
            pltpu.SemaphoreType.DMA,
            pltpu.SemaphoreType.DMA,
            pltpu.SemaphoreType.DMA,
        ],
    )(body)


def _finish_body(f0_ref, f1_ref, s0_ref, s1_ref, feat_ref, misc_ref):
    st = s0_ref[...] + s1_ref[...]
    cnt = st[:, 0:1]
    cN = st[:, 1:2]
    cCA = st[:, 2:3]
    cC = st[:, 3:4]
    cCB = st[:, 4:5]
    dCA = jnp.maximum(cCA, 1.0)
    feat_ref[...] = (f0_ref[...] + f1_ref[...]) / dCA

    pax = st[:, 5:6] / dCA
    pay = st[:, 6:7] / dCA
    paz = st[:, 7:8] / dCA
    dCB = jnp.maximum(cCB, 1.0)
    pbx = st[:, 8:9] / dCB
    pby = st[:, 9:10] / dCB
    pbz = st[:, 10:11] / dCB
    no_cb = (jnp.abs(pbx) + jnp.abs(pby) + jnp.abs(pbz)) < 1e-6
    pbx = jnp.where(no_cb, pax, pbx)
    pby = jnp.where(no_cb, pay, pby)
    pbz = jnp.where(no_cb, paz, pbz)

    e1x = pbx - pax
    e1y = pby - pay
    e1z = pbz - paz
    n1 = jnp.sqrt(e1x * e1x + e1y * e1y + e1z * e1z)
    d1 = jnp.maximum(n1, 1e-6)
    e1ux = e1x / d1
    e1uy = e1y / d1
    e1uz = e1z / d1
    n2a = jnp.sqrt(e1ux * e1ux + e1uy * e1uy)
    use_b = n2a < 1e-6
    e2x = jnp.where(use_b, -e1uz, e1uy)
    e2y = jnp.where(use_b, 0.0, -e1ux)
    e2z = jnp.where(use_b, e1ux, 0.0)
    n2 = jnp.sqrt(e2x * e2x + e2y * e2y + e2z * e2z)
    d2 = jnp.maximum(n2, 1e-6)
    e2ux = e2x / d2
    e2uy = e2y / d2
    e2uz = e2z / d2
    e3x = e1uy * e2uz - e1uz * e2uy
    e3y = e1uz * e2ux - e1ux * e2uz
    e3z = e1ux * e2uy - e1uy * e2ux

    blk = f0_ref.shape[0]
    ridx = (lax.broadcasted_iota(jnp.int32, (blk, 1), 0)
            + pl.program_id(0) * blk)
    valid = (n1 > 1e-6) & (n2 > 1e-6) & (ridx < R - 1)

    maskf = jnp.where((cnt >= 3.0) & (cN > 0.0) & (cCA > 0.0) & (cC > 0.0),
                      1.0, 0.0)
    misc_ref[:, 0:1] = pax
    misc_ref[:, 1:2] = pay
    misc_ref[:, 2:3] = paz
    misc_ref[:, 3:4] = pbx
    misc_ref[:, 4:5] = pby
    misc_ref[:, 5:6] = pbz
    misc_ref[:, 6:7] = jnp.where(valid, e1ux, 1.0)
    misc_ref[:, 7:8] = jnp.where(valid, e2ux, 0.0)
    misc_ref[:, 8:9] = jnp.where(valid, e3x, 0.0)
    misc_ref[:, 9:10] = jnp.where(valid, e1uy, 0.0)
    misc_ref[:, 10:11] = jnp.where(valid, e2uy, 1.0)
    misc_ref[:, 11:12] = jnp.where(valid, e3y, 0.0)
    misc_ref[:, 12:13] = jnp.where(valid, e1uz, 0.0)
    misc_ref[:, 13:14] = jnp.where(valid, e2uz, 0.0)
    misc_ref[:, 14:15] = jnp.where(valid, e3z, 1.0)
    misc_ref[:, 15:16] = maskf


def kernel(node_features, node_positions, atom_type_ids, segment_ids):
    SUB = 2560
    PAD = SUB * 128 - N_AT
    segs1 = segment_ids.astype(jnp.int32)
    types1 = atom_type_ids.astype(jnp.int32)
    segs2 = jnp.pad(segs1, (0, PAD)).reshape(SUB, 128)
    types2 = jnp.pad(types1, (0, PAD), constant_values=21).reshape(SUB, 128)
    pos_t = jnp.pad(node_positions.astype(jnp.float32).T,
                    ((0, 0), (0, PAD))).reshape(3, SUB, 128)

    B2 = 320
    grid0 = SUB // B2
    spec2 = pl.BlockSpec((B2, 128), lambda i: (i, 0))
    prep_out = pl.pallas_call(
        _prep_body,
        grid=(grid0,),
        in_specs=[spec2] * 5,
        out_specs=[spec2] * (1 + NSTAT),
        out_shape=([jax.ShapeDtypeStruct((SUB, 128), jnp.int32)]
                   + [jax.ShapeDtypeStruct((SUB, 128), jnp.float32)] * NSTAT),
    )(segs2, types2, pos_t[0], pos_t[1], pos_t[2])
    adj = prep_out[0]

    stats_wide = jnp.pad(
        jnp.stack(prep_out[1:], axis=0).reshape(NSTAT, SUB * 128)[:, :N_AT].T,
        ((0, 0), (0, H - NSTAT)))
    KR = CHUNK // 128
    adj3 = jnp.pad(adj.reshape(SUB * 128)[:N_AT].reshape(NCHUNKS, KR, 128),
                   ((0, 0), (0, 8 - KR), (0, 0)))
    seg3 = jnp.pad(segs1.reshape(NCHUNKS, KR, 128),
                   ((0, 0), (0, 8 - KR), (0, 0)))
    zf = jnp.zeros((ACC_ROWS, H), jnp.float32)

    feat_part, = _make_sc_scatter(H)(node_features, adj3, zf)
    stat_part, = _make_sc_scatter(H)(stats_wide, seg3, zf)

    f0 = feat_part[:R]
    f1 = feat_part[ACC_ROWS:ACC_ROWS + R]
    s0 = stat_part[:R]
    s1 = stat_part[ACC_ROWS:ACC_ROWS + R]

    BLK = 1000
    grid = R // BLK
    res_feat, misc = pl.pallas_call(
        _finish_body,
        grid=(grid,),
        in_specs=[pl.BlockSpec((BLK, H), lambda i: (i, 0))] * 4,
        out_specs=[
            pl.BlockSpec((BLK, H), lambda i: (i, 0)),
            pl.BlockSpec((BLK, 16), lambda i: (i, 0)),
        ],
        out_shape=[
            jax.ShapeDtypeStruct((R, H), jnp.float32),
            jax.ShapeDtypeStruct((R, 16), jnp.float32),
        ],
    )(f0, f1, s0, s1)

    pos_CA = misc[:, 0:3]
    pos_CB = misc[:, 3:6]
    frames = misc[:, 6:15].reshape(R, 3, 3)
    residue_mask = misc[:, 15] > 0.5
    return (res_feat, pos_CA, pos_CB, frames, segment_ids, residue_mask)
